# baseline (speedup 1.0000x reference)
import functools

import jax
import jax.numpy as jnp
from jax import lax
from jax.experimental import pallas as pl
from jax.experimental.pallas import tpu as pltpu
from jax.experimental.pallas import tpu_sc as plsc

NC = 2
NS = 16
CH = 128
DP = 160


def _matmul_kernel(x_ref, w_ref, b_ref, o_ref):
    h = (jnp.dot(x_ref[...], w_ref[...], preferred_element_type=jnp.float32)
         + b_ref[...])
    bn = h.shape[0]
    o_ref[...] = jnp.concatenate(
        [h, jnp.ones((bn, DP - h.shape[1]), jnp.float32)],
        axis=-1).astype(jnp.bfloat16)


def _finalize_kernel(a_ref, o_ref):
    a = a_ref[0].astype(jnp.float32) + a_ref[1].astype(jnp.float32)
    deg = jnp.maximum(a[:, 128:129], 1.0)
    o_ref[...] = jnp.maximum(a[:, :128] / deg, 0.0)


def _make_sc_agg(n_pad, nch):
    rpt = n_pad // (NS * CH)

    mesh = plsc.VectorSubcoreMesh(core_axis_name="c", subcore_axis_name="s")

    @functools.partial(
        pl.kernel,
        mesh=mesh,
        compiler_params=pltpu.CompilerParams(use_tc_tiling_on_sc=False),
        out_type=jax.ShapeDtypeStruct((NC, n_pad, DP), jnp.bfloat16),
        scratch_types=[
            pltpu.VMEM((nch, CH), jnp.int32),
            pltpu.VMEM((nch, CH), jnp.int32),
            pltpu.VMEM((2, CH, DP), jnp.bfloat16),
            pltpu.VMEM_SHARED((n_pad, DP), jnp.bfloat16),
            pltpu.SemaphoreType.DMA,
            pltpu.SemaphoreType.DMA,
        ],
    )
    def sc_agg(h_hbm, src_hbm, dst_hbm, zrow_hbm,
               agg_out, src_v, dst_v, bufs_v, acc_sh, gsem0, gsem1):
        c = lax.axis_index("c")
        s = lax.axis_index("s")

        def zbody(i, carry):
            r0 = (s * rpt + i) * CH
            pltpu.sync_copy(zrow_hbm, acc_sh.at[pl.ds(r0, CH)])
            return carry

        lax.fori_loop(0, rpt, zbody, 0)

        pltpu.sync_copy(src_hbm.at[c, s], src_v)
        pltpu.sync_copy(dst_hbm.at[c, s], dst_v)
        plsc.subcore_barrier()

        def ebody(t, carry):
            j0 = 2 * t
            pltpu.make_async_copy(h_hbm.at[src_v.at[j0]], bufs_v.at[0],
                                  gsem0).wait()
            pltpu.async_copy(h_hbm.at[src_v.at[j0 + 1]], bufs_v.at[1], gsem1)
            pltpu.sync_copy(bufs_v.at[0], acc_sh.at[dst_v.at[j0]], add=True)

            pltpu.make_async_copy(h_hbm.at[src_v.at[j0 + 1]], bufs_v.at[1],
                                  gsem1).wait()

            @pl.when(t < nch // 2 - 1)
            def _():
                pltpu.async_copy(h_hbm.at[src_v.at[j0 + 2]], bufs_v.at[0],
                                 gsem0)

            pltpu.sync_copy(bufs_v.at[1], acc_sh.at[dst_v.at[j0 + 1]],
                            add=True)
            return carry

        pltpu.async_copy(h_hbm.at[src_v.at[0]], bufs_v.at[0], gsem0)
        lax.fori_loop(0, nch // 2, ebody, 0)
        plsc.subcore_barrier()

        def wbody(i, carry):
            r0 = (s * rpt + i) * CH
            pltpu.sync_copy(acc_sh.at[pl.ds(r0, CH)],
                            agg_out.at[c, pl.ds(r0, CH)])
            return carry

        lax.fori_loop(0, rpt, wbody, 0)

    return sc_agg


def kernel(x, edge_index, W, b):
    n, d_in = x.shape
    d = W.shape[1]
    e = edge_index.shape[1]

    bn = 1000
    h = pl.pallas_call(
        _matmul_kernel,
        grid=(n // bn,),
        in_specs=[
            pl.BlockSpec((bn, d_in), lambda i: (i, 0)),
            pl.BlockSpec((d_in, d), lambda i: (0, 0)),
            pl.BlockSpec((1, d), lambda i: (0, 0)),
        ],
        out_specs=pl.BlockSpec((bn, DP), lambda i: (i, 0)),
        out_shape=jax.ShapeDtypeStruct((n, DP), jnp.bfloat16),
    )(x, W, b.reshape(1, d))

    nw = NC * NS
    nch = -(-e // (nw * CH))
    nch += nch % 2
    e_pad = nw * nch * CH
    n_pad = -(-(n + 1) // (NS * CH)) * (NS * CH)

    src = edge_index[0]
    dst = edge_index[1]
    pad = e_pad - e
    src3 = jnp.concatenate([src, jnp.zeros((pad,), jnp.int32)]).reshape(
        NC, NS, nch, CH)
    dst3 = jnp.concatenate([dst, jnp.full((pad,), n, jnp.int32)]).reshape(
        NC, NS, nch, CH)

    zrow = jnp.zeros((CH, DP), jnp.bfloat16)

    agg_p = _make_sc_agg(n_pad, nch)(h, src3, dst3, zrow)

    out = pl.pallas_call(
        _finalize_kernel,
        grid=(n // bn,),
        in_specs=[
            pl.BlockSpec((NC, bn, DP), lambda i: (0, i, 0)),
        ],
        out_specs=pl.BlockSpec((bn, d), lambda i: (i, 0)),
        out_shape=jax.ShapeDtypeStruct((n, d), jnp.float32),
    )(agg_p)
    return out


# speedup vs baseline: 1.3023x; 1.3023x over previous
import functools

import jax
import jax.numpy as jnp
from jax import lax
from jax.experimental import pallas as pl
from jax.experimental.pallas import tpu as pltpu
from jax.experimental.pallas import tpu_sc as plsc

NC = 2
NS = 16
CH = 128
DP = 160


def _matmul_kernel(x_ref, w_ref, b_ref, o_ref):
    h = (jnp.dot(x_ref[...], w_ref[...], preferred_element_type=jnp.float32)
         + b_ref[...])
    bn = h.shape[0]
    o_ref[...] = jnp.concatenate(
        [h, jnp.ones((bn, DP - h.shape[1]), jnp.float32)],
        axis=-1).astype(jnp.bfloat16)


def _finalize_kernel(a_ref, o_ref):
    a = a_ref[0].astype(jnp.float32) + a_ref[1].astype(jnp.float32)
    deg = jnp.maximum(a[:, 128:129], 1.0)
    o_ref[...] = jnp.maximum(a[:, :128] / deg, 0.0)


def _make_sc_agg(n_pad, nch0, nch1):
    rpt = n_pad // (NS * CH)

    mesh = plsc.VectorSubcoreMesh(core_axis_name="c", subcore_axis_name="s")

    @functools.partial(
        pl.kernel,
        mesh=mesh,
        compiler_params=pltpu.CompilerParams(use_tc_tiling_on_sc=False),
        out_type=jax.ShapeDtypeStruct((NC, n_pad, DP), jnp.bfloat16),
        scratch_types=[
            pltpu.VMEM((nch1, CH), jnp.int32),
            pltpu.VMEM((nch1, CH), jnp.int32),
            pltpu.VMEM((CH, DP), jnp.bfloat16),
            pltpu.VMEM_SHARED((n_pad, DP), jnp.bfloat16),
            pltpu.SemaphoreType.DMA,
        ],
    )
    def sc_agg(h_hbm, src_hbm, dst_hbm, zrow_hbm,
               agg_out, src_v, dst_v, rows_v, acc_sh, sem):
        c = lax.axis_index("c")
        s = lax.axis_index("s")

        def zbody(i, carry):
            r0 = (s * rpt + i) * CH
            pltpu.sync_copy(zrow_hbm, acc_sh.at[pl.ds(r0, CH)])
            return carry

        lax.fori_loop(0, rpt, zbody, 0)

        base = lax.select(c == 0, s * nch0, NS * nch0 + s * nch1)
        mych = lax.select(c == 0, nch0, nch1)
        pltpu.sync_copy(src_hbm.at[pl.ds(base, nch1)], src_v)
        pltpu.sync_copy(dst_hbm.at[pl.ds(base, nch1)], dst_v)
        plsc.subcore_barrier()

        def ebody(j, carry):
            pltpu.async_copy(h_hbm.at[src_v.at[j]], rows_v, sem).wait()
            pltpu.sync_copy(rows_v, acc_sh.at[dst_v.at[j]], add=True)
            return carry

        lax.fori_loop(0, mych, ebody, 0)
        plsc.subcore_barrier()

        def wbody(i, carry):
            r0 = (s * rpt + i) * CH
            pltpu.sync_copy(acc_sh.at[pl.ds(r0, CH)],
                            agg_out.at[c, pl.ds(r0, CH)])
            return carry

        lax.fori_loop(0, rpt, wbody, 0)

    return sc_agg


def kernel(x, edge_index, W, b):
    n, d_in = x.shape
    d = W.shape[1]
    e = edge_index.shape[1]

    bn = 1000
    h = pl.pallas_call(
        _matmul_kernel,
        grid=(n // bn,),
        in_specs=[
            pl.BlockSpec((bn, d_in), lambda i: (i, 0)),
            pl.BlockSpec((d_in, d), lambda i: (0, 0)),
            pl.BlockSpec((1, d), lambda i: (0, 0)),
        ],
        out_specs=pl.BlockSpec((bn, DP), lambda i: (i, 0)),
        out_shape=jax.ShapeDtypeStruct((n, DP), jnp.bfloat16),
    )(x, W, b.reshape(1, d))

    nchp = -(-e // (NS * CH))
    nch0 = max(1, round(0.39 * nchp))
    nch1 = nchp - nch0
    e_pad = NS * nchp * CH
    n_pad = -(-(n + 1) // (NS * CH)) * (NS * CH)

    src = edge_index[0]
    dst = edge_index[1]
    pad = e_pad - e
    src3 = jnp.concatenate([src, jnp.zeros((pad,), jnp.int32)]).reshape(
        NS * nchp, CH)
    dst3 = jnp.concatenate([dst, jnp.full((pad,), n, jnp.int32)]).reshape(
        NS * nchp, CH)

    zrow = jnp.zeros((CH, DP), jnp.bfloat16)

    agg_p = _make_sc_agg(n_pad, nch0, nch1)(h, src3, dst3, zrow)

    out = pl.pallas_call(
        _finalize_kernel,
        grid=(n // bn,),
        in_specs=[
            pl.BlockSpec((NC, bn, DP), lambda i: (0, i, 0)),
        ],
        out_specs=pl.BlockSpec((bn, d), lambda i: (i, 0)),
        out_shape=jax.ShapeDtypeStruct((n, d), jnp.float32),
    )(agg_p)
    return out


# speedup vs baseline: 1.5744x; 1.2090x over previous
import functools

import jax
import jax.numpy as jnp
from jax import lax
from jax.experimental import pallas as pl
from jax.experimental.pallas import tpu as pltpu
from jax.experimental.pallas import tpu_sc as plsc

NC = 2
NS = 16
CH = 128
DP = 160


def _matmul_kernel(x_ref, w_ref, b_ref, o_ref):
    h = (jnp.dot(x_ref[...], w_ref[...], preferred_element_type=jnp.float32)
         + b_ref[...])
    bn = h.shape[0]
    o_ref[...] = jnp.concatenate(
        [h, jnp.ones((bn, DP - h.shape[1]), jnp.float32)],
        axis=-1).astype(jnp.bfloat16)


def _finalize_kernel(a_ref, o_ref):
    a = a_ref[0].astype(jnp.float32) + a_ref[1].astype(jnp.float32)
    deg = jnp.maximum(a[:, 128:129], 1.0)
    o_ref[...] = jnp.maximum(a[:, :128] / deg, 0.0)


def _make_sc_agg(n_pad, tch, nch0, nch1, rem):
    rpt = n_pad // (NS * CH)
    nmx = max(nch0, nch1 + 1)

    mesh = plsc.VectorSubcoreMesh(core_axis_name="c", subcore_axis_name="s")

    @functools.partial(
        pl.kernel,
        mesh=mesh,
        compiler_params=pltpu.CompilerParams(use_tc_tiling_on_sc=False),
        out_type=jax.ShapeDtypeStruct((NC, n_pad, DP), jnp.bfloat16),
        scratch_types=[
            pltpu.VMEM((nmx, CH), jnp.int32),
            pltpu.VMEM((nmx, CH), jnp.int32),
            pltpu.VMEM((CH, DP), jnp.bfloat16),
            pltpu.VMEM_SHARED((n_pad, DP), jnp.bfloat16),
            pltpu.SemaphoreType.DMA,
        ],
    )
    def sc_agg(h_hbm, src_hbm, dst_hbm, zrow_hbm,
               agg_out, src_v, dst_v, rows_v, acc_sh, sem):
        c = lax.axis_index("c")
        s = lax.axis_index("s")

        def zbody(i, carry):
            r0 = (s * rpt + i) * CH
            pltpu.sync_copy(zrow_hbm, acc_sh.at[pl.ds(r0, CH)])
            return carry

        lax.fori_loop(0, rpt, zbody, 0)

        base = jnp.where(c == 0,
                         s * nch0,
                         NS * nch0 + s * nch1 + jnp.minimum(s, rem))
        mych = jnp.where(c == 0, nch0, nch1 + (s < rem).astype(jnp.int32))
        base2 = jnp.minimum(base, tch - nmx)
        off = base - base2
        pltpu.sync_copy(src_hbm.at[pl.ds(base2, nmx)], src_v)
        pltpu.sync_copy(dst_hbm.at[pl.ds(base2, nmx)], dst_v)
        plsc.subcore_barrier()

        def ebody(j, carry):
            pltpu.async_copy(h_hbm.at[src_v.at[off + j]], rows_v, sem).wait()
            pltpu.sync_copy(rows_v, acc_sh.at[dst_v.at[off + j]], add=True)
            return carry

        lax.fori_loop(0, mych, ebody, 0)
        plsc.subcore_barrier()

        def wbody(i, carry):
            r0 = (s * rpt + i) * CH
            pltpu.sync_copy(acc_sh.at[pl.ds(r0, CH)],
                            agg_out.at[c, pl.ds(r0, CH)])
            return carry

        lax.fori_loop(0, rpt, wbody, 0)

    return sc_agg


def kernel(x, edge_index, W, b):
    n, d_in = x.shape
    d = W.shape[1]
    e = edge_index.shape[1]

    bn = 1000
    h = pl.pallas_call(
        _matmul_kernel,
        grid=(n // bn,),
        in_specs=[
            pl.BlockSpec((bn, d_in), lambda i: (i, 0)),
            pl.BlockSpec((d_in, d), lambda i: (0, 0)),
            pl.BlockSpec((1, d), lambda i: (0, 0)),
        ],
        out_specs=pl.BlockSpec((bn, DP), lambda i: (i, 0)),
        out_shape=jax.ShapeDtypeStruct((n, DP), jnp.bfloat16),
    )(x, W, b.reshape(1, d))

    tch = -(-e // CH)
    nch0 = round(tch * 0.522 / NS)
    nch1 = (tch - NS * nch0) // NS
    rem = tch - NS * (nch0 + nch1)
    n_pad = -(-(n + 1) // (NS * CH)) * (NS * CH)

    src = edge_index[0]
    dst = edge_index[1]
    if tch * CH > e:
        pad = tch * CH - e
        src = jnp.concatenate([src, jnp.zeros((pad,), jnp.int32)])
        dst = jnp.concatenate([dst, jnp.full((pad,), n, jnp.int32)])
    src3 = src.reshape(tch, CH)
    dst3 = dst.reshape(tch, CH)

    zrow = jnp.zeros((CH, DP), jnp.bfloat16)

    agg_p = _make_sc_agg(n_pad, tch, nch0, nch1, rem)(h, src3, dst3, zrow)

    out = pl.pallas_call(
        _finalize_kernel,
        grid=(n // bn,),
        in_specs=[
            pl.BlockSpec((NC, bn, DP), lambda i: (0, i, 0)),
        ],
        out_specs=pl.BlockSpec((bn, d), lambda i: (i, 0)),
        out_shape=jax.ShapeDtypeStruct((n, d), jnp.float32),
    )(agg_p)
    return out


# speedup vs baseline: 1.6792x; 1.0666x over previous
import functools

import jax
import jax.numpy as jnp
from jax import lax
from jax.experimental import pallas as pl
from jax.experimental.pallas import tpu as pltpu
from jax.experimental.pallas import tpu_sc as plsc

NC = 2
NS = 16
CH = 128
DP = 160


def _matmul_kernel(x_ref, w_ref, b_ref, o_ref):
    h = (jnp.dot(x_ref[...], w_ref[...], preferred_element_type=jnp.float32)
         + b_ref[...])
    bn = h.shape[0]
    o_ref[...] = jnp.concatenate(
        [h, jnp.ones((bn, DP - h.shape[1]), jnp.float32)],
        axis=-1).astype(jnp.bfloat16)


def _finalize_kernel(a_ref, o_ref):
    a = a_ref[0].astype(jnp.float32) + a_ref[1].astype(jnp.float32)
    deg = jnp.maximum(a[:, 128:129], 1.0)
    o_ref[...] = jnp.maximum(a[:, :128] / deg, 0.0)


def _make_sc_agg(n_pad, tch, nch0, nch1, rem):
    rpt = n_pad // (NS * CH)
    nmx = max(nch0, nch1 + 1)

    mesh = plsc.VectorSubcoreMesh(core_axis_name="c", subcore_axis_name="s")

    @functools.partial(
        pl.kernel,
        mesh=mesh,
        compiler_params=pltpu.CompilerParams(use_tc_tiling_on_sc=False),
        out_type=jax.ShapeDtypeStruct((NC, n_pad, DP), jnp.bfloat16),
        scratch_types=[
            pltpu.VMEM((nmx, CH), jnp.int32),
            pltpu.VMEM((nmx, CH), jnp.int32),
            pltpu.VMEM((CH, DP), jnp.bfloat16),
            pltpu.VMEM_SHARED((n_pad, DP), jnp.bfloat16),
            pltpu.SemaphoreType.DMA,
        ],
    )
    def sc_agg(h_hbm, edge_hbm, zrow_hbm,
               agg_out, src_v, dst_v, rows_v, acc_sh, sem):
        c = lax.axis_index("c")
        s = lax.axis_index("s")

        def zbody(i, carry):
            r0 = (s * rpt + i) * CH
            pltpu.sync_copy(zrow_hbm, acc_sh.at[pl.ds(r0, CH)])
            return carry

        lax.fori_loop(0, rpt, zbody, 0)

        base = jnp.where(c == 0,
                         s * nch0,
                         NS * nch0 + s * nch1 + jnp.minimum(s, rem))
        mych = jnp.where(c == 0, nch0, nch1 + (s < rem).astype(jnp.int32))
        base2 = jnp.minimum(base, tch - nmx)
        off = base - base2
        pltpu.sync_copy(edge_hbm.at[0, pl.ds(base2, nmx)], src_v)
        pltpu.sync_copy(edge_hbm.at[1, pl.ds(base2, nmx)], dst_v)
        plsc.subcore_barrier()

        def ebody(j, carry):
            pltpu.async_copy(h_hbm.at[src_v.at[off + j]], rows_v, sem).wait()
            pltpu.sync_copy(rows_v, acc_sh.at[dst_v.at[off + j]], add=True)
            return carry

        lax.fori_loop(0, mych, ebody, 0)
        plsc.subcore_barrier()

        def wbody(i, carry):
            r0 = (s * rpt + i) * CH
            pltpu.sync_copy(acc_sh.at[pl.ds(r0, CH)],
                            agg_out.at[c, pl.ds(r0, CH)])
            return carry

        lax.fori_loop(0, rpt, wbody, 0)

    return sc_agg


def kernel(x, edge_index, W, b):
    n, d_in = x.shape
    d = W.shape[1]
    e = edge_index.shape[1]

    bn = 1000
    h = pl.pallas_call(
        _matmul_kernel,
        grid=(n // bn,),
        in_specs=[
            pl.BlockSpec((bn, d_in), lambda i: (i, 0)),
            pl.BlockSpec((d_in, d), lambda i: (0, 0)),
            pl.BlockSpec((1, d), lambda i: (0, 0)),
        ],
        out_specs=pl.BlockSpec((bn, DP), lambda i: (i, 0)),
        out_shape=jax.ShapeDtypeStruct((n, DP), jnp.bfloat16),
    )(x, W, b.reshape(1, d))

    tch = -(-e // CH)
    nch0 = round(tch * 0.5 / NS)
    nch1 = (tch - NS * nch0) // NS
    rem = tch - NS * (nch0 + nch1)
    n_pad = -(-(n + 1) // (NS * CH)) * (NS * CH)

    ei = edge_index
    if tch * CH > e:
        pad = tch * CH - e
        ei = jnp.concatenate(
            [ei, jnp.stack([jnp.zeros((pad,), jnp.int32),
                            jnp.full((pad,), n, jnp.int32)])], axis=1)
    edges3 = ei.reshape(2, tch, CH)

    zrow = jnp.zeros((CH, DP), jnp.bfloat16)

    agg_p = _make_sc_agg(n_pad, tch, nch0, nch1, rem)(h, edges3, zrow)

    out = pl.pallas_call(
        _finalize_kernel,
        grid=(n // bn,),
        in_specs=[
            pl.BlockSpec((NC, bn, DP), lambda i: (0, i, 0)),
        ],
        out_specs=pl.BlockSpec((bn, d), lambda i: (i, 0)),
        out_shape=jax.ShapeDtypeStruct((n, d), jnp.float32),
    )(agg_p)
    return out


# speedup vs baseline: 1.8482x; 1.1006x over previous
import functools

import jax
import jax.numpy as jnp
from jax import lax
from jax.experimental import pallas as pl
from jax.experimental.pallas import tpu as pltpu
from jax.experimental.pallas import tpu_sc as plsc

NC = 2
NS = 16
CH = 128
GB = 4
DP = 160


def _matmul_kernel(x_ref, w_ref, b_ref, o_ref):
    h = (jnp.dot(x_ref[...], w_ref[...], preferred_element_type=jnp.float32)
         + b_ref[...])
    bn = h.shape[0]
    o_ref[...] = jnp.concatenate(
        [h, jnp.ones((bn, DP - h.shape[1]), jnp.float32)],
        axis=-1).astype(jnp.bfloat16)


def _finalize_kernel(a_ref, o_ref):
    a = a_ref[0].astype(jnp.float32) + a_ref[1].astype(jnp.float32)
    deg = jnp.maximum(a[:, 128:129], 1.0)
    o_ref[...] = jnp.maximum(a[:, :128] / deg, 0.0)


def _make_sc_agg(n_pad, tch, nch0, nch1, rem):
    rpt = n_pad // (NS * CH)
    nmx = max(nch0, nch1 + 1)

    mesh = plsc.VectorSubcoreMesh(core_axis_name="c", subcore_axis_name="s")

    @functools.partial(
        pl.kernel,
        mesh=mesh,
        compiler_params=pltpu.CompilerParams(use_tc_tiling_on_sc=False),
        out_type=jax.ShapeDtypeStruct((NC, n_pad, DP), jnp.bfloat16),
        scratch_types=[
            pltpu.VMEM((nmx * CH,), jnp.int32),
            pltpu.VMEM((nmx, CH), jnp.int32),
            pltpu.VMEM((GB * CH, DP), jnp.bfloat16),
            pltpu.VMEM_SHARED((n_pad, DP), jnp.bfloat16),
            pltpu.SemaphoreType.DMA,
        ],
    )
    def sc_agg(h_hbm, src_hbm, dst_hbm, zrow_hbm,
               agg_out, src_v, dst_v, rows_v, acc_sh, sem):
        c = lax.axis_index("c")
        s = lax.axis_index("s")

        def zbody(i, carry):
            r0 = (s * rpt + i) * CH
            pltpu.sync_copy(zrow_hbm, acc_sh.at[pl.ds(r0, CH)])
            return carry

        lax.fori_loop(0, rpt, zbody, 0)

        base = jnp.where(c == 0,
                         s * nch0,
                         NS * nch0 + s * nch1 + jnp.minimum(s, rem))
        mych = jnp.where(c == 0, nch0, nch1 + (s < rem).astype(jnp.int32))
        base2 = jnp.minimum(base, tch - nmx)
        off = base - base2
        pltpu.sync_copy(src_hbm.at[pl.ds(base2 * CH, nmx * CH)], src_v)
        pltpu.sync_copy(dst_hbm.at[pl.ds(base2, nmx)], dst_v)
        plsc.subcore_barrier()

        ngrp = mych // GB

        def ebody(g, carry):
            j = off + g * GB
            pltpu.async_copy(h_hbm.at[src_v.at[pl.ds(j * CH, GB * CH)]],
                             rows_v, sem).wait()
            for i in range(GB):
                pltpu.sync_copy(rows_v.at[pl.ds(i * CH, CH)],
                                acc_sh.at[dst_v.at[j + i]], add=True)
            return carry

        lax.fori_loop(0, ngrp, ebody, 0)

        def tbody(j, carry):
            pltpu.async_copy(
                h_hbm.at[src_v.at[pl.ds((off + j) * CH, CH)]],
                rows_v.at[pl.ds(0, CH)], sem).wait()
            pltpu.sync_copy(rows_v.at[pl.ds(0, CH)],
                            acc_sh.at[dst_v.at[off + j]], add=True)
            return carry

        lax.fori_loop(ngrp * GB, mych, tbody, 0)
        plsc.subcore_barrier()

        def wbody(i, carry):
            r0 = (s * rpt + i) * CH
            pltpu.sync_copy(acc_sh.at[pl.ds(r0, CH)],
                            agg_out.at[c, pl.ds(r0, CH)])
            return carry

        lax.fori_loop(0, rpt, wbody, 0)

    return sc_agg


def kernel(x, edge_index, W, b):
    n, d_in = x.shape
    d = W.shape[1]
    e = edge_index.shape[1]

    bn = 1000
    h = pl.pallas_call(
        _matmul_kernel,
        grid=(n // bn,),
        in_specs=[
            pl.BlockSpec((bn, d_in), lambda i: (i, 0)),
            pl.BlockSpec((d_in, d), lambda i: (0, 0)),
            pl.BlockSpec((1, d), lambda i: (0, 0)),
        ],
        out_specs=pl.BlockSpec((bn, DP), lambda i: (i, 0)),
        out_shape=jax.ShapeDtypeStruct((n, DP), jnp.bfloat16),
    )(x, W, b.reshape(1, d))

    tch = -(-e // CH)
    nch0 = round(tch * 0.5 / NS)
    nch1 = (tch - NS * nch0) // NS
    rem = tch - NS * (nch0 + nch1)
    n_pad = -(-(n + 1) // (NS * CH)) * (NS * CH)

    ei = edge_index
    if tch * CH > e:
        pad = tch * CH - e
        ei = jnp.concatenate(
            [ei, jnp.stack([jnp.zeros((pad,), jnp.int32),
                            jnp.full((pad,), n, jnp.int32)])], axis=1)
    src_flat = ei[0]
    dst3 = ei[1].reshape(tch, CH)

    zrow = jnp.zeros((CH, DP), jnp.bfloat16)

    agg_p = _make_sc_agg(n_pad, tch, nch0, nch1, rem)(h, src_flat, dst3,
                                                      zrow)

    out = pl.pallas_call(
        _finalize_kernel,
        grid=(n // bn,),
        in_specs=[
            pl.BlockSpec((NC, bn, DP), lambda i: (0, i, 0)),
        ],
        out_specs=pl.BlockSpec((bn, d), lambda i: (i, 0)),
        out_shape=jax.ShapeDtypeStruct((n, d), jnp.float32),
    )(agg_p)
    return out


# speedup vs baseline: 1.8604x; 1.0066x over previous
import functools

import jax
import jax.numpy as jnp
from jax import lax
from jax.experimental import pallas as pl
from jax.experimental.pallas import tpu as pltpu
from jax.experimental.pallas import tpu_sc as plsc

NC = 2
NS = 16
CH = 128
GB = 4
SB = 2
DP = 160


def _matmul_kernel(x_ref, w_ref, b_ref, o_ref):
    h = (jnp.dot(x_ref[...], w_ref[...], preferred_element_type=jnp.float32)
         + b_ref[...])
    bn = h.shape[0]
    o_ref[...] = jnp.concatenate(
        [h, jnp.ones((bn, DP - h.shape[1]), jnp.float32)],
        axis=-1).astype(jnp.bfloat16)


def _finalize_kernel(a_ref, o_ref):
    a = a_ref[0].astype(jnp.float32) + a_ref[1].astype(jnp.float32)
    deg = jnp.maximum(a[:, 128:129], 1.0)
    o_ref[...] = jnp.maximum(a[:, :128] / deg, 0.0)


def _make_sc_agg(n_pad, tch, nch0, nch1, rem):
    rpt = n_pad // (NS * CH)
    nmx = max(nch0, nch1 + 1)

    mesh = plsc.VectorSubcoreMesh(core_axis_name="c", subcore_axis_name="s")

    @functools.partial(
        pl.kernel,
        mesh=mesh,
        compiler_params=pltpu.CompilerParams(use_tc_tiling_on_sc=False),
        out_type=jax.ShapeDtypeStruct((NC, n_pad, DP), jnp.bfloat16),
        scratch_types=[
            pltpu.VMEM((nmx * CH,), jnp.int32),
            pltpu.VMEM((nmx * CH,), jnp.int32),
            pltpu.VMEM((GB * CH, DP), jnp.bfloat16),
            pltpu.VMEM_SHARED((n_pad, DP), jnp.bfloat16),
            pltpu.SemaphoreType.DMA,
        ],
    )
    def sc_agg(h_hbm, src_hbm, dst_hbm, zrow_hbm,
               agg_out, src_v, dst_v, rows_v, acc_sh, sem):
        c = lax.axis_index("c")
        s = lax.axis_index("s")

        def zbody(i, carry):
            r0 = (s * rpt + i) * CH
            pltpu.sync_copy(zrow_hbm, acc_sh.at[pl.ds(r0, CH)])
            return carry

        lax.fori_loop(0, rpt, zbody, 0)

        base = jnp.where(c == 0,
                         s * nch0,
                         NS * nch0 + s * nch1 + jnp.minimum(s, rem))
        mych = jnp.where(c == 0, nch0, nch1 + (s < rem).astype(jnp.int32))
        base2 = jnp.minimum(base, tch - nmx)
        off = base - base2
        pltpu.sync_copy(src_hbm.at[pl.ds(base2 * CH, nmx * CH)], src_v)
        pltpu.sync_copy(dst_hbm.at[pl.ds(base2 * CH, nmx * CH)], dst_v)
        plsc.subcore_barrier()

        ngrp = mych // GB

        def ebody(g, carry):
            j = off + g * GB
            pltpu.async_copy(h_hbm.at[src_v.at[pl.ds(j * CH, GB * CH)]],
                             rows_v, sem).wait()
            for i in range(GB // SB):
                pltpu.sync_copy(
                    rows_v.at[pl.ds(i * SB * CH, SB * CH)],
                    acc_sh.at[dst_v.at[pl.ds((j + i * SB) * CH, SB * CH)]],
                    add=True)
            return carry

        lax.fori_loop(0, ngrp, ebody, 0)

        def tbody(j, carry):
            pltpu.async_copy(
                h_hbm.at[src_v.at[pl.ds((off + j) * CH, CH)]],
                rows_v.at[pl.ds(0, CH)], sem).wait()
            pltpu.sync_copy(
                rows_v.at[pl.ds(0, CH)],
                acc_sh.at[dst_v.at[pl.ds((off + j) * CH, CH)]], add=True)
            return carry

        lax.fori_loop(ngrp * GB, mych, tbody, 0)
        plsc.subcore_barrier()

        def wbody(i, carry):
            r0 = (s * rpt + i) * CH
            pltpu.sync_copy(acc_sh.at[pl.ds(r0, CH)],
                            agg_out.at[c, pl.ds(r0, CH)])
            return carry

        lax.fori_loop(0, rpt, wbody, 0)

    return sc_agg


def kernel(x, edge_index, W, b):
    n, d_in = x.shape
    d = W.shape[1]
    e = edge_index.shape[1]

    bn = 1000
    h = pl.pallas_call(
        _matmul_kernel,
        grid=(n // bn,),
        in_specs=[
            pl.BlockSpec((bn, d_in), lambda i: (i, 0)),
            pl.BlockSpec((d_in, d), lambda i: (0, 0)),
            pl.BlockSpec((1, d), lambda i: (0, 0)),
        ],
        out_specs=pl.BlockSpec((bn, DP), lambda i: (i, 0)),
        out_shape=jax.ShapeDtypeStruct((n, DP), jnp.bfloat16),
    )(x, W, b.reshape(1, d))

    tch = -(-e // CH)
    nch0 = round(tch * 0.5 / NS)
    nch1 = (tch - NS * nch0) // NS
    rem = tch - NS * (nch0 + nch1)
    n_pad = -(-(n + 1) // (NS * CH)) * (NS * CH)

    ei = edge_index
    if tch * CH > e:
        pad = tch * CH - e
        ei = jnp.concatenate(
            [ei, jnp.stack([jnp.zeros((pad,), jnp.int32),
                            jnp.full((pad,), n, jnp.int32)])], axis=1)
    zrow = jnp.zeros((CH, DP), jnp.bfloat16)

    agg_p = _make_sc_agg(n_pad, tch, nch0, nch1, rem)(h, ei[0], ei[1], zrow)

    out = pl.pallas_call(
        _finalize_kernel,
        grid=(n // bn,),
        in_specs=[
            pl.BlockSpec((NC, bn, DP), lambda i: (0, i, 0)),
        ],
        out_specs=pl.BlockSpec((bn, d), lambda i: (i, 0)),
        out_shape=jax.ShapeDtypeStruct((n, d), jnp.float32),
    )(agg_p)
    return out


# speedup vs baseline: 2.1652x; 1.1638x over previous
import functools

import jax
import jax.numpy as jnp
from jax import lax
from jax.experimental import pallas as pl
from jax.experimental.pallas import tpu as pltpu
from jax.experimental.pallas import tpu_sc as plsc

NC = 2
NS = 16
CH = 128
GB = 2
SB = 2
DP = 160


def _matmul_kernel(x_ref, w_ref, b_ref, o_ref):
    h = (jnp.dot(x_ref[...], w_ref[...], preferred_element_type=jnp.float32)
         + b_ref[...])
    bn = h.shape[0]
    o_ref[...] = jnp.concatenate(
        [h, jnp.ones((bn, DP - h.shape[1]), jnp.float32)],
        axis=-1).astype(jnp.bfloat16)


def _finalize_kernel(a_ref, o_ref):
    a = a_ref[0].astype(jnp.float32) + a_ref[1].astype(jnp.float32)
    deg = jnp.maximum(a[:, 128:129], 1.0)
    o_ref[...] = jnp.maximum(a[:, :128] / deg, 0.0)


def _make_sc_agg(n_pad, tch, nch0, nch1, rem):
    rpt = n_pad // (NS * CH)
    nmx = max(nch0, nch1 + 1)

    mesh = plsc.VectorSubcoreMesh(core_axis_name="c", subcore_axis_name="s")

    @functools.partial(
        pl.kernel,
        mesh=mesh,
        compiler_params=pltpu.CompilerParams(use_tc_tiling_on_sc=False),
        out_type=jax.ShapeDtypeStruct((NC, n_pad, DP), jnp.bfloat16),
        scratch_types=[
            pltpu.VMEM((nmx * CH,), jnp.int32),
            pltpu.VMEM((nmx * CH,), jnp.int32),
            pltpu.VMEM((2, GB * CH, DP), jnp.bfloat16),
            pltpu.VMEM_SHARED((n_pad, DP), jnp.bfloat16),
            pltpu.SemaphoreType.DMA,
            pltpu.SemaphoreType.DMA,
        ],
    )
    def sc_agg(h_hbm, src_hbm, dst_hbm, zrow_hbm,
               agg_out, src_v, dst_v, rows_v, acc_sh, sema, semb):
        c = lax.axis_index("c")
        s = lax.axis_index("s")

        def zbody(i, carry):
            r0 = (s * rpt + i) * CH
            pltpu.sync_copy(zrow_hbm, acc_sh.at[pl.ds(r0, CH)])
            return carry

        lax.fori_loop(0, rpt, zbody, 0)

        base = jnp.where(c == 0,
                         s * nch0,
                         NS * nch0 + s * nch1 + jnp.minimum(s, rem))
        mych = jnp.where(c == 0, nch0, nch1 + (s < rem).astype(jnp.int32))
        base2 = jnp.minimum(base, tch - nmx)
        off = base - base2
        pltpu.sync_copy(src_hbm.at[pl.ds(base2 * CH, nmx * CH)], src_v)
        pltpu.sync_copy(dst_hbm.at[pl.ds(base2 * CH, nmx * CH)], dst_v)
        plsc.subcore_barrier()

        ngrp = mych // GB

        def fire(g, b, sem):
            pltpu.async_copy(
                h_hbm.at[src_v.at[pl.ds((off + g * GB) * CH, GB * CH)]],
                rows_v.at[b], sem)

        def drain(g, b, sem):
            pltpu.make_async_copy(
                h_hbm.at[src_v.at[pl.ds((off + g * GB) * CH, GB * CH)]],
                rows_v.at[b], sem).wait()

        def scat(g, b):
            j = off + g * GB
            for i in range(GB // SB):
                pltpu.sync_copy(
                    rows_v.at[b].at[pl.ds(i * SB * CH, SB * CH)],
                    acc_sh.at[dst_v.at[pl.ds((j + i * SB) * CH, SB * CH)]],
                    add=True)

        def ebody(t, carry):
            g0 = 2 * t
            g1 = 2 * t + 1

            @pl.when(g1 < ngrp)
            def _():
                fire(g1, 1, semb)

            drain(g0, 0, sema)
            scat(g0, 0)

            @pl.when(g1 < ngrp)
            def _():
                @pl.when(g1 + 1 < ngrp)
                def _():
                    fire(g1 + 1, 0, sema)

                drain(g1, 1, semb)
                scat(g1, 1)

            return carry

        @pl.when(ngrp > 0)
        def _():
            fire(0, 0, sema)

        lax.fori_loop(0, (ngrp + 1) // 2, ebody, 0)

        def tbody(j, carry):
            pltpu.async_copy(
                h_hbm.at[src_v.at[pl.ds((off + j) * CH, CH)]],
                rows_v.at[0].at[pl.ds(0, CH)], sema).wait()
            pltpu.sync_copy(
                rows_v.at[0].at[pl.ds(0, CH)],
                acc_sh.at[dst_v.at[pl.ds((off + j) * CH, CH)]], add=True)
            return carry

        lax.fori_loop(ngrp * GB, mych, tbody, 0)
        plsc.subcore_barrier()

        def wbody(i, carry):
            r0 = (s * rpt + i) * CH
            pltpu.sync_copy(acc_sh.at[pl.ds(r0, CH)],
                            agg_out.at[c, pl.ds(r0, CH)])
            return carry

        lax.fori_loop(0, rpt, wbody, 0)

    return sc_agg


def kernel(x, edge_index, W, b):
    n, d_in = x.shape
    d = W.shape[1]
    e = edge_index.shape[1]

    bn = 1000
    h = pl.pallas_call(
        _matmul_kernel,
        grid=(n // bn,),
        in_specs=[
            pl.BlockSpec((bn, d_in), lambda i: (i, 0)),
            pl.BlockSpec((d_in, d), lambda i: (0, 0)),
            pl.BlockSpec((1, d), lambda i: (0, 0)),
        ],
        out_specs=pl.BlockSpec((bn, DP), lambda i: (i, 0)),
        out_shape=jax.ShapeDtypeStruct((n, DP), jnp.bfloat16),
    )(x, W, b.reshape(1, d))

    tch = -(-e // CH)
    nch0 = round(tch * 0.5 / NS)
    nch1 = (tch - NS * nch0) // NS
    rem = tch - NS * (nch0 + nch1)
    n_pad = -(-(n + 1) // (NS * CH)) * (NS * CH)

    ei = edge_index
    if tch * CH > e:
        pad = tch * CH - e
        ei = jnp.concatenate(
            [ei, jnp.stack([jnp.zeros((pad,), jnp.int32),
                            jnp.full((pad,), n, jnp.int32)])], axis=1)
    zrow = jnp.zeros((CH, DP), jnp.bfloat16)

    agg_p = _make_sc_agg(n_pad, tch, nch0, nch1, rem)(h, ei[0], ei[1], zrow)

    out = pl.pallas_call(
        _finalize_kernel,
        grid=(n // bn,),
        in_specs=[
            pl.BlockSpec((NC, bn, DP), lambda i: (0, i, 0)),
        ],
        out_specs=pl.BlockSpec((bn, d), lambda i: (i, 0)),
        out_shape=jax.ShapeDtypeStruct((n, d), jnp.float32),
    )(agg_p)
    return out
